# baseline (speedup 1.0000x reference)
import functools

import jax
import jax.numpy as jnp
from jax import lax
from jax.experimental import pallas as pl
from jax.experimental.pallas import tpu as pltpu
from jax.experimental.pallas import tpu_sc as plsc

B = 4096
D = 512
R = 6
S = 8

B_TC = 1024
B_SC = B - B_TC

_info = plsc.get_sparse_core_info()
NC = _info.num_cores
NS = _info.num_subcores
L = _info.num_lanes
NW = NC * NS
BPW = B_SC // NW
CH = BPW // 2
NCH = BPW // CH
NG = (R * D) // L

TBLK = 512

_mesh = plsc.VectorSubcoreMesh(core_axis_name="c", subcore_axis_name="s")


@functools.partial(
    pl.kernel,
    mesh=_mesh,
    compiler_params=pltpu.CompilerParams(needs_layout_passes=False),
    out_type=jax.ShapeDtypeStruct((B_SC, 1, D), jnp.float32),
    scratch_types=[
        pltpu.VMEM((BPW + L,), jnp.int32),
        pltpu.VMEM((NG // NS * L * S,), jnp.float32),
        pltpu.VMEM((R * D,), jnp.float32),
        pltpu.VMEM_SHARED((R * D,), jnp.float32),
        pltpu.VMEM((2, CH, D), jnp.float32),
        pltpu.SemaphoreType.DMA,
        pltpu.SemaphoreType.DMA,
        pltpu.SemaphoreType.DMA,
        pltpu.SemaphoreType.DMA,
        pltpu.SemaphoreType.DMA,
        pltpu.SemaphoreType.DMA,
    ],
)
def _sc_walk(z_hbm, idx_hbm, w_hbm, out_hbm,
             idx_v, w_v, wsum_v, wsum_sh, z_v,
             sem_i, sem_w, sem_z0, sem_z1, sem_o0, sem_o1):
    sem_z = [sem_z0, sem_z1]
    sem_o = [sem_o0, sem_o1]
    sid = lax.axis_index("s")
    wid = sid * NC + lax.axis_index("c")
    base = wid * BPW
    GPT = NG // NS
    SLC = GPT * L * S

    cp_i = pltpu.async_copy(idx_hbm.at[pl.ds(base, BPW)], idx_v.at[pl.ds(0, BPW)],
                            sem_i)
    cp_w = pltpu.async_copy(w_hbm.at[pl.ds(sid * SLC, SLC)], w_v, sem_w)
    zload = [None, None]
    zload[0] = pltpu.async_copy(z_hbm.at[pl.ds(base, CH), 0], z_v.at[0], sem_z[0])

    iota = lax.iota(jnp.int32, L)
    cp_w.wait()

    for g in range(GPT):
        a0 = g * (L * S) + iota * S
        gs = [plsc.load_gather(w_v, [a0 + s]) for s in range(S)]
        while len(gs) > 1:
            gs = [gs[i] + gs[i + 1] for i in range(0, len(gs), 2)]
        wsum_v[pl.ds(g * L, L)] = gs[0]
    pltpu.sync_copy(wsum_v.at[pl.ds(0, GPT * L)],
                    wsum_sh.at[pl.ds(sid * GPT * L, GPT * L)])
    plsc.subcore_barrier()
    pltpu.sync_copy(wsum_sh, wsum_v)
    cp_i.wait()

    outw = [None, None]
    for ci in range(NCH):
        buf = ci % 2
        nbuf = (ci + 1) % 2
        if ci + 1 < NCH:
            if outw[nbuf] is not None:
                outw[nbuf].wait()
                outw[nbuf] = None
            zload[nbuf] = pltpu.async_copy(
                z_hbm.at[pl.ds(base + (ci + 1) * CH, CH), 0], z_v.at[nbuf],
                sem_z[nbuf])
        zload[buf].wait()

        @plsc.parallel_loop(0, CH)
        def row_body(r, ci=ci, buf=buf):
            tb = idx_v[pl.ds(ci * CH + r, L)][0] * D
            wvs = [wsum_v[pl.ds(tb + c * L, L)] for c in range(D // L)]
            for c in range(D // L):
                plsc.addupdate(z_v.at[buf, r, pl.ds(c * L, L)], wvs[c])
        outw[buf] = pltpu.async_copy(
            z_v.at[buf], out_hbm.at[pl.ds(base + ci * CH, CH), 0], sem_o[buf])
    for w in outw:
        if w is not None:
            w.wait()


def _tc_body(idx_ref, z_ref, w_ref, o_ref):
    idxb = idx_ref[0, 0, :]
    wsum = jnp.sum(w_ref[:, 0, :, :], axis=-1)
    onehot = (idxb[:, None]
              == lax.broadcasted_iota(jnp.int32, (TBLK, R), 1)
              ).astype(jnp.float32)
    o_ref[:, 0, :] = z_ref[:, 0, :] + jnp.dot(
        onehot, wsum, preferred_element_type=jnp.float32)


_OFF = B_SC // TBLK


def _tc_tail(idx3, z, w):
    return pl.pallas_call(
        _tc_body,
        grid=(B_TC // TBLK,),
        in_specs=[
            pl.BlockSpec((1, 1, TBLK), lambda i: (i + _OFF, 0, 0)),
            pl.BlockSpec((TBLK, 1, D), lambda i: (i + _OFF, 0, 0)),
            pl.BlockSpec((R, 1, D, S), lambda i: (0, 0, 0, 0)),
        ],
        out_specs=pl.BlockSpec((TBLK, 1, D), lambda i: (i, 0, 0)),
        out_shape=jax.ShapeDtypeStruct((B_TC, 1, D), jnp.float32),
    )(idx3, z, w)


def kernel(z, alpha, index_, w_blondhair):
    idx = index_.astype(jnp.int32)
    sc_out = _sc_walk(z, idx, w_blondhair.reshape(R * D * S))
    tc_out = _tc_tail(idx.reshape(B // TBLK, 1, TBLK), z, w_blondhair)
    return jnp.concatenate([sc_out, tc_out], axis=0)


# speedup vs baseline: 1.9699x; 1.9699x over previous
import functools

import jax
import jax.numpy as jnp
from jax import lax
from jax.experimental import pallas as pl
from jax.experimental.pallas import tpu as pltpu
from jax.experimental.pallas import tpu_sc as plsc

B = 4096
D = 512
R = 6
S = 8

_info = plsc.get_sparse_core_info()
NC = _info.num_cores
NS = _info.num_subcores
L = _info.num_lanes
NW = NC * NS
BPW = B // NW
CH = 32
NCH = BPW // CH
NG = (R * D) // L

_mesh = plsc.VectorSubcoreMesh(core_axis_name="c", subcore_axis_name="s")


@functools.partial(
    pl.kernel,
    mesh=_mesh,
    compiler_params=pltpu.CompilerParams(needs_layout_passes=False),
    out_type=jax.ShapeDtypeStruct((B, 1, D), jnp.float32),
    scratch_types=[
        pltpu.VMEM((BPW + L,), jnp.int32),
        pltpu.VMEM((NG // NS * L * S,), jnp.float32),
        pltpu.VMEM((R * D,), jnp.float32),
        pltpu.VMEM_SHARED((R * D,), jnp.float32),
        pltpu.VMEM((2, CH, D), jnp.float32),
        pltpu.SemaphoreType.DMA,
        pltpu.SemaphoreType.DMA,
        pltpu.SemaphoreType.DMA,
        pltpu.SemaphoreType.DMA,
        pltpu.SemaphoreType.DMA,
        pltpu.SemaphoreType.DMA,
    ],
)
def _sc_walk(z_hbm, idx_hbm, w_hbm, out_hbm,
             idx_v, w_v, wsum_v, wsum_sh, z_v,
             sem_i, sem_w, sem_z0, sem_z1, sem_o0, sem_o1):
    sem_z = [sem_z0, sem_z1]
    sem_o = [sem_o0, sem_o1]
    sid = lax.axis_index("s")
    wid = sid * NC + lax.axis_index("c")
    base = wid * BPW
    GPT = NG // NS
    SLC = GPT * L * S

    cp_i = pltpu.async_copy(idx_hbm.at[pl.ds(base, BPW)], idx_v.at[pl.ds(0, BPW)],
                            sem_i)
    cp_w = pltpu.async_copy(w_hbm.at[pl.ds(sid * SLC, SLC)], w_v, sem_w)
    zload = [None, None]
    zload[0] = pltpu.async_copy(z_hbm.at[pl.ds(base, CH), 0], z_v.at[0], sem_z[0])

    iota = lax.iota(jnp.int32, L)
    cp_w.wait()

    for g in range(GPT):
        a0 = g * (L * S) + iota * S
        gs = [plsc.load_gather(w_v, [a0 + s]) for s in range(S)]
        while len(gs) > 1:
            gs = [gs[i] + gs[i + 1] for i in range(0, len(gs), 2)]
        wsum_v[pl.ds(g * L, L)] = gs[0]
    pltpu.sync_copy(wsum_v.at[pl.ds(0, GPT * L)],
                    wsum_sh.at[pl.ds(sid * GPT * L, GPT * L)])
    plsc.subcore_barrier()
    pltpu.sync_copy(wsum_sh, wsum_v)
    cp_i.wait()

    outw = [None, None]
    for ci in range(NCH):
        buf = ci % 2
        nbuf = (ci + 1) % 2
        if ci + 1 < NCH:
            if outw[nbuf] is not None:
                outw[nbuf].wait()
                outw[nbuf] = None
            zload[nbuf] = pltpu.async_copy(
                z_hbm.at[pl.ds(base + (ci + 1) * CH, CH), 0], z_v.at[nbuf],
                sem_z[nbuf])
        zload[buf].wait()

        @plsc.parallel_loop(0, CH)
        def row_body(r, ci=ci, buf=buf):
            tb = idx_v[pl.ds(ci * CH + r, L)][0] * D
            wvs = [wsum_v[pl.ds(tb + c * L, L)] for c in range(D // L)]
            for c in range(D // L):
                plsc.addupdate(z_v.at[buf, r, pl.ds(c * L, L)], wvs[c])
        outw[buf] = pltpu.async_copy(
            z_v.at[buf], out_hbm.at[pl.ds(base + ci * CH, CH), 0], sem_o[buf])
    for w in outw:
        if w is not None:
            w.wait()


def kernel(z, alpha, index_, w_blondhair):
    idx = index_.astype(jnp.int32)
    return _sc_walk(z, idx, w_blondhair.reshape(R * D * S))


# speedup vs baseline: 2.1026x; 1.0674x over previous
import functools

import jax
import jax.numpy as jnp
from jax import lax
from jax.experimental import pallas as pl
from jax.experimental.pallas import tpu as pltpu
from jax.experimental.pallas import tpu_sc as plsc

B = 4096
D = 512
R = 6
S = 8

_info = plsc.get_sparse_core_info()
NC = _info.num_cores
NS = _info.num_subcores
L = _info.num_lanes
NW = NC * NS
BPW = B // NW
CH = 64
NCH = BPW // CH
NG = (R * D) // L

_mesh = plsc.VectorSubcoreMesh(core_axis_name="c", subcore_axis_name="s")


@functools.partial(
    pl.kernel,
    mesh=_mesh,
    compiler_params=pltpu.CompilerParams(needs_layout_passes=False),
    out_type=jax.ShapeDtypeStruct((B, 1, D), jnp.float32),
    scratch_types=[
        pltpu.VMEM((BPW + L,), jnp.int32),
        pltpu.VMEM((NG // NS * L * S,), jnp.float32),
        pltpu.VMEM((R * D,), jnp.float32),
        pltpu.VMEM_SHARED((R * D,), jnp.float32),
        pltpu.VMEM((2, CH, D), jnp.float32),
        pltpu.SemaphoreType.DMA,
        pltpu.SemaphoreType.DMA,
        pltpu.SemaphoreType.DMA,
        pltpu.SemaphoreType.DMA,
        pltpu.SemaphoreType.DMA,
        pltpu.SemaphoreType.DMA,
    ],
)
def _sc_walk(z_hbm, idx_hbm, w_hbm, out_hbm,
             idx_v, w_v, wsum_v, wsum_sh, z_v,
             sem_i, sem_w, sem_z0, sem_z1, sem_o0, sem_o1):
    sem_z = [sem_z0, sem_z1]
    sem_o = [sem_o0, sem_o1]
    sid = lax.axis_index("s")
    wid = sid * NC + lax.axis_index("c")
    base = wid * BPW
    GPT = NG // NS
    SLC = GPT * L * S

    cp_i = pltpu.async_copy(idx_hbm.at[pl.ds(base, BPW)], idx_v.at[pl.ds(0, BPW)],
                            sem_i)
    cp_w = pltpu.async_copy(w_hbm.at[pl.ds(sid * SLC, SLC)], w_v, sem_w)
    zload = [None, None]
    zload[0] = pltpu.async_copy(z_hbm.at[pl.ds(base, CH), 0], z_v.at[0], sem_z[0])

    iota = lax.iota(jnp.int32, L)
    cp_w.wait()

    for g in range(GPT):
        a0 = g * (L * S) + iota * S
        gs = [plsc.load_gather(w_v, [a0 + s]) for s in range(S)]
        while len(gs) > 1:
            gs = [gs[i] + gs[i + 1] for i in range(0, len(gs), 2)]
        wsum_v[pl.ds(g * L, L)] = gs[0]
    pltpu.sync_copy(wsum_v.at[pl.ds(0, GPT * L)],
                    wsum_sh.at[pl.ds(sid * GPT * L, GPT * L)])
    plsc.subcore_barrier()
    pltpu.sync_copy(wsum_sh, wsum_v)
    cp_i.wait()

    outw = [None, None]
    for ci in range(NCH):
        buf = ci % 2
        nbuf = (ci + 1) % 2
        if ci + 1 < NCH:
            if outw[nbuf] is not None:
                outw[nbuf].wait()
                outw[nbuf] = None
            zload[nbuf] = pltpu.async_copy(
                z_hbm.at[pl.ds(base + (ci + 1) * CH, CH), 0], z_v.at[nbuf],
                sem_z[nbuf])
        zload[buf].wait()

        @plsc.parallel_loop(0, CH)
        def row_body(r, ci=ci, buf=buf):
            tb = idx_v[pl.ds(ci * CH + r, L)][0] * D
            wvs = [wsum_v[pl.ds(tb + c * L, L)] for c in range(D // L)]
            for c in range(D // L):
                plsc.addupdate(z_v.at[buf, r, pl.ds(c * L, L)], wvs[c])
        outw[buf] = pltpu.async_copy(
            z_v.at[buf], out_hbm.at[pl.ds(base + ci * CH, CH), 0], sem_o[buf])
    for w in outw:
        if w is not None:
            w.wait()


def kernel(z, alpha, index_, w_blondhair):
    idx = index_.astype(jnp.int32)
    return _sc_walk(z, idx, w_blondhair.reshape(R * D * S))
